# speedup vs baseline: 9.7294x; 9.7294x over previous
import functools

import jax
import jax.numpy as jnp
from jax import lax
from jax.experimental import pallas as pl
from jax.experimental.pallas import tpu as pltpu
from jax.experimental.pallas import tpu_sc as plsc

N = 10000
E = 160000
D_IN = 256
HID = 512
HID2 = 256
NCLS = 16

NP = 10240
F = 128
NS = 16
NC = 2
ROWS_PER_TILE = NP // NS
CHUNK = 80
EDGES_PER_TILE = E // NS
DEG_W = NC * NS
DEG_EDGES = E // DEG_W
DEG_CHUNK = 40


def _vsmesh():
    return plsc.VectorSubcoreMesh(core_axis_name="c", subcore_axis_name="s")


def _sc_degree(dst):
    @functools.partial(
        pl.kernel,
        out_type=jax.ShapeDtypeStruct((NC, NP), jnp.float32),
        mesh=_vsmesh(),
        scratch_types=[
            pltpu.VMEM((DEG_CHUNK,), jnp.int32),
            pltpu.VMEM((DEG_CHUNK,), jnp.float32),
            pltpu.VMEM((ROWS_PER_TILE,), jnp.float32),
            pltpu.VMEM_SHARED((NP,), jnp.float32),
        ],
    )
    def k(dst_h, out_h, idxb, ones, zbuf, acc):
        c = lax.axis_index("c")
        s = lax.axis_index("s")
        wid = s * NC + c

        def fill_z(i, _):
            zbuf[pl.ds(i * 16, 16)] = jnp.zeros((16,), jnp.float32)
            return 0

        lax.fori_loop(0, ROWS_PER_TILE // 16, fill_z, 0)
        one16 = jnp.ones((16,), jnp.float32)
        ones[pl.ds(0, 16)] = one16
        ones[pl.ds(16, 16)] = one16
        ones[pl.ds(24, 16)] = one16

        z0 = s * ROWS_PER_TILE
        pltpu.sync_copy(zbuf, acc.at[pl.ds(z0, ROWS_PER_TILE)])
        plsc.subcore_barrier()

        base = wid * DEG_EDGES

        def body(j, _):
            pltpu.sync_copy(dst_h.at[pl.ds(base + j * DEG_CHUNK, DEG_CHUNK)], idxb)
            pltpu.sync_copy(ones, acc.at[idxb], add=True)
            return 0

        lax.fori_loop(0, DEG_EDGES // DEG_CHUNK, body, 0)
        plsc.subcore_barrier()
        pltpu.sync_copy(acc.at[pl.ds(z0, ROWS_PER_TILE)],
                        out_h.at[c, pl.ds(z0, ROWS_PER_TILE)])

    return k(dst)


def _sc_aggregate(table, src, dst):
    @functools.partial(
        pl.kernel,
        out_type=jax.ShapeDtypeStruct((NC, NP, F), jnp.float32),
        mesh=_vsmesh(),
        scratch_types=[
            pltpu.VMEM((CHUNK,), jnp.int32),
            pltpu.VMEM((CHUNK,), jnp.int32),
            pltpu.VMEM((CHUNK, F), jnp.float32),
            pltpu.VMEM_SHARED((NP, F), jnp.float32),
            pltpu.SemaphoreType.DMA,
        ],
    )
    def k(table_h, src_h, dst_h, out_h, sidx, didx, rows, acc, sem):
        c = lax.axis_index("c")
        s = lax.axis_index("s")
        r0 = s * ROWS_PER_TILE
        pltpu.sync_copy(table_h.at[pl.ds(c * NP + r0, ROWS_PER_TILE)],
                        acc.at[pl.ds(r0, ROWS_PER_TILE)])
        plsc.subcore_barrier()

        base = s * EDGES_PER_TILE

        def body(j, _):
            eb = base + j * CHUNK
            pltpu.sync_copy(src_h.at[pl.ds(eb, CHUNK)], sidx)
            pltpu.sync_copy(dst_h.at[pl.ds(eb, CHUNK)], didx)

            def adj(i, _):
                sidx[pl.ds(i * 16, 16)] = sidx[pl.ds(i * 16, 16)] + c * NP
                return 0

            lax.fori_loop(0, CHUNK // 16, adj, 0)
            pltpu.async_copy(table_h.at[sidx], rows, sem).wait()
            pltpu.sync_copy(rows, acc.at[didx], add=True)
            return 0

        lax.fori_loop(0, EDGES_PER_TILE // CHUNK, body, 0)
        plsc.subcore_barrier()
        pltpu.sync_copy(acc.at[pl.ds(r0, ROWS_PER_TILE)],
                        out_h.at[c, pl.ds(r0, ROWS_PER_TILE)])

    return k(table, src, dst)


_RB = 1024
_GRID = NP // _RB


def _tc_scale(degp, xpad):
    def body(degp_ref, x_ref, xs2_ref, dinv_ref):
        deg = degp_ref[0] + degp_ref[1] + 1.0
        di = lax.rsqrt(deg)
        dinv_ref[...] = di[:, None]
        xs = x_ref[...] * di[:, None]
        xs2_ref[0] = xs[:, :F]
        xs2_ref[1] = xs[:, F:]

    return pl.pallas_call(
        body,
        grid=(_GRID,),
        in_specs=[
            pl.BlockSpec((NC, _RB), lambda b: (0, b)),
            pl.BlockSpec((_RB, D_IN), lambda b: (b, 0)),
        ],
        out_specs=[
            pl.BlockSpec((NC, _RB, F), lambda b: (0, b, 0)),
            pl.BlockSpec((_RB, 1), lambda b: (b, 0)),
        ],
        out_shape=[
            jax.ShapeDtypeStruct((NC, NP, F), jnp.float32),
            jax.ShapeDtypeStruct((NP, 1), jnp.float32),
        ],
    )(degp, xpad)


def _tc_mid(agg1, dinv, W1, b1, W2):
    def body(agg_ref, dinv_ref, w1_ref, b1_ref, w2_ref, out_ref):
        di = dinv_ref[...]
        a = jnp.concatenate([agg_ref[0], agg_ref[1]], axis=1) * di
        h = jnp.dot(a, w1_ref[...], preferred_element_type=jnp.float32)
        h = jnp.maximum(h + b1_ref[...][None, :], 0.0)
        m = jnp.dot(h, w2_ref[...], preferred_element_type=jnp.float32) * di
        out_ref[0] = m[:, :F]
        out_ref[1] = m[:, F:]

    return pl.pallas_call(
        body,
        grid=(_GRID,),
        in_specs=[
            pl.BlockSpec((NC, _RB, F), lambda b: (0, b, 0)),
            pl.BlockSpec((_RB, 1), lambda b: (b, 0)),
            pl.BlockSpec((D_IN, HID), lambda b: (0, 0)),
            pl.BlockSpec((HID,), lambda b: (0,)),
            pl.BlockSpec((HID, HID2), lambda b: (0, 0)),
        ],
        out_specs=pl.BlockSpec((NC, _RB, F), lambda b: (0, b, 0)),
        out_shape=jax.ShapeDtypeStruct((NC, NP, F), jnp.float32),
    )(agg1, dinv, W1, b1, W2)


def _tc_final(agg2, dinv, b2, Wc, bc):
    def body(agg_ref, dinv_ref, b2_ref, wc_ref, bc_ref, out_ref):
        di = dinv_ref[...]
        a = jnp.concatenate([agg_ref[0], agg_ref[1]], axis=1) * di
        z = jnp.maximum(a + b2_ref[...][None, :], 0.0)
        logits = jnp.dot(z, wc_ref[...], preferred_element_type=jnp.float32)
        logits = logits + bc_ref[...][None, :]
        m = jnp.max(logits, axis=1, keepdims=True)
        lse = m + jnp.log(jnp.sum(jnp.exp(logits - m), axis=1, keepdims=True))
        out_ref[...] = logits - lse

    return pl.pallas_call(
        body,
        grid=(_GRID,),
        in_specs=[
            pl.BlockSpec((NC, _RB, F), lambda b: (0, b, 0)),
            pl.BlockSpec((_RB, 1), lambda b: (b, 0)),
            pl.BlockSpec((HID2,), lambda b: (0,)),
            pl.BlockSpec((HID2, NCLS), lambda b: (0, 0)),
            pl.BlockSpec((NCLS,), lambda b: (0,)),
        ],
        out_specs=pl.BlockSpec((_RB, NCLS), lambda b: (b, 0)),
        out_shape=jax.ShapeDtypeStruct((NP, NCLS), jnp.float32),
    )(agg2, dinv, b2, Wc, bc)


def kernel(x, edge_index, W1, b1, W2, b2, Wc, bc):
    src = edge_index[0]
    dst = edge_index[1]
    degp = _sc_degree(dst)
    xpad = jnp.pad(x, ((0, NP - N), (0, 0)))
    xs2, dinv = _tc_scale(degp, xpad)
    agg1 = _sc_aggregate(xs2.reshape(NC * NP, F), src, dst)
    ms2 = _tc_mid(agg1, dinv, W1, b1, W2)
    agg2 = _sc_aggregate(ms2.reshape(NC * NP, F), src, dst)
    out = _tc_final(agg2, dinv, b2, Wc, bc)
    return out[:N]


# speedup vs baseline: 15.6765x; 1.6112x over previous
import functools

import jax
import jax.numpy as jnp
from jax import lax
from jax.experimental import pallas as pl
from jax.experimental.pallas import tpu as pltpu
from jax.experimental.pallas import tpu_sc as plsc

N = 10000
E = 160000
D_IN = 256
HID = 512
HID2 = 256
NCLS = 16

NP = 10240
F = 128
NS = 16
NC = 2
ROWS_PER_TILE = NP // NS
CHUNK = 80
EDGES_PER_TILE = E // NS
DEG_W = NC * NS
DEG_EDGES = E // DEG_W
DEG_CHUNK = 40


def _vsmesh():
    return plsc.VectorSubcoreMesh(core_axis_name="c", subcore_axis_name="s")


def _sc_degree(dst):
    @functools.partial(
        pl.kernel,
        out_type=jax.ShapeDtypeStruct((NC, NP), jnp.float32),
        mesh=_vsmesh(),
        scratch_types=[
            pltpu.VMEM((DEG_CHUNK,), jnp.int32),
            pltpu.VMEM((DEG_CHUNK,), jnp.float32),
            pltpu.VMEM((ROWS_PER_TILE,), jnp.float32),
            pltpu.VMEM_SHARED((NP,), jnp.float32),
        ],
    )
    def k(dst_h, out_h, idxb, ones, zbuf, acc):
        c = lax.axis_index("c")
        s = lax.axis_index("s")
        wid = s * NC + c

        def fill_z(i, _):
            zbuf[pl.ds(i * 16, 16)] = jnp.zeros((16,), jnp.float32)
            return 0

        lax.fori_loop(0, ROWS_PER_TILE // 16, fill_z, 0)
        one16 = jnp.ones((16,), jnp.float32)
        ones[pl.ds(0, 16)] = one16
        ones[pl.ds(16, 16)] = one16
        ones[pl.ds(24, 16)] = one16

        z0 = s * ROWS_PER_TILE
        pltpu.sync_copy(zbuf, acc.at[pl.ds(z0, ROWS_PER_TILE)])
        plsc.subcore_barrier()

        base = wid * DEG_EDGES

        def body(j, _):
            pltpu.sync_copy(dst_h.at[pl.ds(base + j * DEG_CHUNK, DEG_CHUNK)], idxb)
            pltpu.sync_copy(ones, acc.at[idxb], add=True)
            return 0

        lax.fori_loop(0, DEG_EDGES // DEG_CHUNK, body, 0)
        plsc.subcore_barrier()
        pltpu.sync_copy(acc.at[pl.ds(z0, ROWS_PER_TILE)],
                        out_h.at[c, pl.ds(z0, ROWS_PER_TILE)])

    return k(dst)


NBUF = 4
NCHUNK = EDGES_PER_TILE // CHUNK
NGROUP = NCHUNK // NBUF
NTAIL = NCHUNK - NGROUP * NBUF


def _sc_aggregate(tlo, thi, src, dst):
    @functools.partial(
        pl.kernel,
        out_type=jax.ShapeDtypeStruct((NC, NP, F), jnp.float32),
        mesh=_vsmesh(),
        scratch_types=[
            [pltpu.VMEM((CHUNK,), jnp.int32) for _ in range(NBUF)],
            [pltpu.VMEM((CHUNK,), jnp.int32) for _ in range(NBUF)],
            [pltpu.VMEM((CHUNK, F), jnp.float32) for _ in range(NBUF)],
            pltpu.VMEM_SHARED((NP, F), jnp.float32),
            [pltpu.SemaphoreType.DMA for _ in range(NBUF)],
            [pltpu.SemaphoreType.DMA for _ in range(NBUF)],
            [pltpu.SemaphoreType.DMA for _ in range(NBUF)],
        ],
    )
    def k(tlo_h, thi_h, src_h, dst_h, out_h, sidx, didx, rows, acc,
          isem, gsem, ssem):
        c = lax.axis_index("c")
        s = lax.axis_index("s")
        r0 = s * ROWS_PER_TILE
        base = s * EDGES_PER_TILE

        def idx_load(j, p):
            pltpu.async_copy(src_h.at[pl.ds(base + j * CHUNK, CHUNK)],
                             sidx[p], isem[p])
            pltpu.async_copy(dst_h.at[pl.ds(base + j * CHUNK, CHUNK)],
                             didx[p], isem[p])

        def idx_wait(p):
            pltpu.make_async_copy(src_h.at[pl.ds(0, CHUNK)], sidx[p],
                                  isem[p]).wait()
            pltpu.make_async_copy(dst_h.at[pl.ds(0, CHUNK)], didx[p],
                                  isem[p]).wait()

        def run(table_h):
            pltpu.sync_copy(table_h.at[pl.ds(r0, ROWS_PER_TILE)],
                            acc.at[pl.ds(r0, ROWS_PER_TILE)])
            plsc.subcore_barrier()
            for p in range(NBUF):
                idx_load(p, p)
            for p in range(NBUF):
                idx_wait(p)
                pltpu.async_copy(table_h.at[sidx[p]], rows[p], gsem[p])

            def group(g, _):
                for p in range(NBUF):
                    pltpu.make_async_copy(table_h.at[sidx[p]], rows[p],
                                          gsem[p]).wait()
                    pltpu.sync_copy(rows[p], acc.at[didx[p]], add=True)

                @pl.when(g < NGROUP - 1)
                def _():
                    for p in range(NBUF):
                        idx_load((g + 1) * NBUF + p, p)
                    for p in range(NBUF):
                        idx_wait(p)
                        pltpu.async_copy(table_h.at[sidx[p]], rows[p],
                                         gsem[p])
                return 0

            lax.fori_loop(0, NGROUP, group, 0)
            for t in range(NTAIL):
                j = NGROUP * NBUF + t
                idx_load(j, 0)
                idx_wait(0)
                pltpu.async_copy(table_h.at[sidx[0]], rows[0], gsem[0]).wait()
                pltpu.sync_copy(rows[0], acc.at[didx[0]], add=True)
            plsc.subcore_barrier()
            pltpu.sync_copy(acc.at[pl.ds(r0, ROWS_PER_TILE)],
                            out_h.at[c, pl.ds(r0, ROWS_PER_TILE)])

        @pl.when(c == 0)
        def _():
            run(tlo_h)

        @pl.when(c == 1)
        def _():
            run(thi_h)

    return k(tlo, thi, src, dst)


_RB = 1024
_GRID = NP // _RB


def _tc_scale(degp, xpad):
    def body(degp_ref, x_ref, xs2_ref, dinv_ref):
        deg = degp_ref[0] + degp_ref[1] + 1.0
        di = lax.rsqrt(deg)
        dinv_ref[...] = di[:, None]
        xs = x_ref[...] * di[:, None]
        xs2_ref[0] = xs[:, :F]
        xs2_ref[1] = xs[:, F:]

    return pl.pallas_call(
        body,
        grid=(_GRID,),
        in_specs=[
            pl.BlockSpec((NC, _RB), lambda b: (0, b)),
            pl.BlockSpec((_RB, D_IN), lambda b: (b, 0)),
        ],
        out_specs=[
            pl.BlockSpec((NC, _RB, F), lambda b: (0, b, 0)),
            pl.BlockSpec((_RB, 1), lambda b: (b, 0)),
        ],
        out_shape=[
            jax.ShapeDtypeStruct((NC, NP, F), jnp.float32),
            jax.ShapeDtypeStruct((NP, 1), jnp.float32),
        ],
    )(degp, xpad)


def _tc_mid(agg1, dinv, W1, b1, W2):
    def body(agg_ref, dinv_ref, w1_ref, b1_ref, w2_ref, out_ref):
        di = dinv_ref[...]
        a = jnp.concatenate([agg_ref[0], agg_ref[1]], axis=1) * di
        h = jnp.dot(a, w1_ref[...], preferred_element_type=jnp.float32)
        h = jnp.maximum(h + b1_ref[...][None, :], 0.0)
        m = jnp.dot(h, w2_ref[...], preferred_element_type=jnp.float32) * di
        out_ref[0] = m[:, :F]
        out_ref[1] = m[:, F:]

    return pl.pallas_call(
        body,
        grid=(_GRID,),
        in_specs=[
            pl.BlockSpec((NC, _RB, F), lambda b: (0, b, 0)),
            pl.BlockSpec((_RB, 1), lambda b: (b, 0)),
            pl.BlockSpec((D_IN, HID), lambda b: (0, 0)),
            pl.BlockSpec((HID,), lambda b: (0,)),
            pl.BlockSpec((HID, HID2), lambda b: (0, 0)),
        ],
        out_specs=pl.BlockSpec((NC, _RB, F), lambda b: (0, b, 0)),
        out_shape=jax.ShapeDtypeStruct((NC, NP, F), jnp.float32),
    )(agg1, dinv, W1, b1, W2)


def _tc_final(agg2, dinv, b2, Wc, bc):
    def body(agg_ref, dinv_ref, b2_ref, wc_ref, bc_ref, out_ref):
        di = dinv_ref[...]
        a = jnp.concatenate([agg_ref[0], agg_ref[1]], axis=1) * di
        z = jnp.maximum(a + b2_ref[...][None, :], 0.0)
        logits = jnp.dot(z, wc_ref[...], preferred_element_type=jnp.float32)
        logits = logits + bc_ref[...][None, :]
        m = jnp.max(logits, axis=1, keepdims=True)
        lse = m + jnp.log(jnp.sum(jnp.exp(logits - m), axis=1, keepdims=True))
        out_ref[...] = logits - lse

    return pl.pallas_call(
        body,
        grid=(_GRID,),
        in_specs=[
            pl.BlockSpec((NC, _RB, F), lambda b: (0, b, 0)),
            pl.BlockSpec((_RB, 1), lambda b: (b, 0)),
            pl.BlockSpec((HID2,), lambda b: (0,)),
            pl.BlockSpec((HID2, NCLS), lambda b: (0, 0)),
            pl.BlockSpec((NCLS,), lambda b: (0,)),
        ],
        out_specs=pl.BlockSpec((_RB, NCLS), lambda b: (b, 0)),
        out_shape=jax.ShapeDtypeStruct((NP, NCLS), jnp.float32),
    )(agg2, dinv, b2, Wc, bc)


def kernel(x, edge_index, W1, b1, W2, b2, Wc, bc):
    src = edge_index[0]
    dst = edge_index[1]
    degp = _sc_degree(dst)
    xpad = jnp.pad(x, ((0, NP - N), (0, 0)))
    xs2, dinv = _tc_scale(degp, xpad)
    agg1 = _sc_aggregate(xs2[0], xs2[1], src, dst)
    ms2 = _tc_mid(agg1, dinv, W1, b1, W2)
    agg2 = _sc_aggregate(ms2[0], ms2[1], src, dst)
    out = _tc_final(agg2, dinv, b2, Wc, bc)
    return out[:N]


# speedup vs baseline: 17.2076x; 1.0977x over previous
import functools

import jax
import jax.numpy as jnp
from jax import lax
from jax.experimental import pallas as pl
from jax.experimental.pallas import tpu as pltpu
from jax.experimental.pallas import tpu_sc as plsc

N = 10000
E = 160000
D_IN = 256
HID = 512
HID2 = 256
NCLS = 16

NP = 10240
F = 128
NS = 16
NC = 2
ROWS_PER_TILE = NP // NS
CHUNK = 80
EDGES_PER_TILE = E // NS
DEG_W = NC * NS
DEG_EDGES = E // DEG_W
DEG_CHUNK = 40


def _vsmesh():
    return plsc.VectorSubcoreMesh(core_axis_name="c", subcore_axis_name="s")


def _sc_degree(dst):
    @functools.partial(
        pl.kernel,
        out_type=jax.ShapeDtypeStruct((NC, NP), jnp.float32),
        mesh=_vsmesh(),
        scratch_types=[
            pltpu.VMEM((DEG_CHUNK,), jnp.int32),
            pltpu.VMEM((DEG_CHUNK,), jnp.float32),
            pltpu.VMEM((ROWS_PER_TILE,), jnp.float32),
            pltpu.VMEM_SHARED((NP,), jnp.float32),
        ],
    )
    def k(dst_h, out_h, idxb, ones, zbuf, acc):
        c = lax.axis_index("c")
        s = lax.axis_index("s")
        wid = s * NC + c

        def fill_z(i, _):
            zbuf[pl.ds(i * 16, 16)] = jnp.zeros((16,), jnp.float32)
            return 0

        lax.fori_loop(0, ROWS_PER_TILE // 16, fill_z, 0)
        one16 = jnp.ones((16,), jnp.float32)
        ones[pl.ds(0, 16)] = one16
        ones[pl.ds(16, 16)] = one16
        ones[pl.ds(24, 16)] = one16

        z0 = s * ROWS_PER_TILE
        pltpu.sync_copy(zbuf, acc.at[pl.ds(z0, ROWS_PER_TILE)])
        plsc.subcore_barrier()

        base = wid * DEG_EDGES

        def body(j, _):
            pltpu.sync_copy(dst_h.at[pl.ds(base + j * DEG_CHUNK, DEG_CHUNK)], idxb)
            pltpu.sync_copy(ones, acc.at[idxb], add=True)
            return 0

        lax.fori_loop(0, DEG_EDGES // DEG_CHUNK, body, 0)
        plsc.subcore_barrier()
        pltpu.sync_copy(acc.at[pl.ds(z0, ROWS_PER_TILE)],
                        out_h.at[c, pl.ds(z0, ROWS_PER_TILE)])

    return k(dst)


NBUF = 4
NCHUNK = EDGES_PER_TILE // CHUNK
NGROUP = NCHUNK // NBUF
NTAIL = NCHUNK - NGROUP * NBUF


def _sc_aggregate(tlo, thi, src, dst):
    @functools.partial(
        pl.kernel,
        out_type=jax.ShapeDtypeStruct((NC, NP, F), jnp.float32),
        mesh=_vsmesh(),
        scratch_types=[
            [pltpu.VMEM((CHUNK,), jnp.int32) for _ in range(NBUF)],
            [pltpu.VMEM((CHUNK,), jnp.int32) for _ in range(NBUF)],
            [pltpu.VMEM((CHUNK, F), jnp.float32) for _ in range(NBUF)],
            pltpu.VMEM_SHARED((NP, F), jnp.float32),
            [pltpu.SemaphoreType.DMA for _ in range(NBUF)],
            [pltpu.SemaphoreType.DMA for _ in range(NBUF)],
            [pltpu.SemaphoreType.DMA for _ in range(NBUF)],
        ],
    )
    def k(tlo_h, thi_h, src_h, dst_h, out_h, sidx, didx, rows, acc,
          isem, gsem, ssem):
        c = lax.axis_index("c")
        s = lax.axis_index("s")
        r0 = s * ROWS_PER_TILE
        base = s * EDGES_PER_TILE

        def idx_load(j, p):
            pltpu.async_copy(src_h.at[pl.ds(base + j * CHUNK, CHUNK)],
                             sidx[p], isem[p])
            pltpu.async_copy(dst_h.at[pl.ds(base + j * CHUNK, CHUNK)],
                             didx[p], isem[p])

        def idx_wait(p):
            pltpu.make_async_copy(src_h.at[pl.ds(0, CHUNK)], sidx[p],
                                  isem[p]).wait()
            pltpu.make_async_copy(dst_h.at[pl.ds(0, CHUNK)], didx[p],
                                  isem[p]).wait()

        def run(table_h):
            pltpu.sync_copy(table_h.at[pl.ds(r0, ROWS_PER_TILE)],
                            acc.at[pl.ds(r0, ROWS_PER_TILE)])
            plsc.subcore_barrier()
            for p in range(NBUF):
                idx_load(p, p)
            for p in range(NBUF):
                idx_wait(p)
                pltpu.async_copy(table_h.at[sidx[p]], rows[p], gsem[p])

            def scat_drain(p):
                pltpu.make_async_copy(table_h.at[sidx[p]], rows[p],
                                      ssem[p]).wait()

            def group(g, _):
                for p in range(NBUF):
                    pltpu.make_async_copy(table_h.at[sidx[p]], rows[p],
                                          gsem[p]).wait()
                    pltpu.async_copy(rows[p], acc.at[didx[p]], ssem[p],
                                     add=True)

                @pl.when(g < NGROUP - 1)
                def _():
                    for p in range(NBUF):
                        scat_drain(p)
                        idx_load((g + 1) * NBUF + p, p)
                    for p in range(NBUF):
                        idx_wait(p)
                        pltpu.async_copy(table_h.at[sidx[p]], rows[p],
                                         gsem[p])
                return 0

            lax.fori_loop(0, NGROUP, group, 0)
            for p in range(NBUF):
                scat_drain(p)
            for t in range(NTAIL):
                j = NGROUP * NBUF + t
                idx_load(j, 0)
                idx_wait(0)
                pltpu.async_copy(table_h.at[sidx[0]], rows[0], gsem[0]).wait()
                pltpu.sync_copy(rows[0], acc.at[didx[0]], add=True)
            plsc.subcore_barrier()
            pltpu.sync_copy(acc.at[pl.ds(r0, ROWS_PER_TILE)],
                            out_h.at[c, pl.ds(r0, ROWS_PER_TILE)])

        @pl.when(c == 0)
        def _():
            run(tlo_h)

        @pl.when(c == 1)
        def _():
            run(thi_h)

    return k(tlo, thi, src, dst)


_RB = 1024
_GRID = NP // _RB


def _tc_scale(degp, xpad):
    def body(degp_ref, x_ref, xs2_ref, dinv_ref):
        deg = degp_ref[0] + degp_ref[1] + 1.0
        di = lax.rsqrt(deg)
        dinv_ref[...] = di[:, None]
        xs = x_ref[...] * di[:, None]
        xs2_ref[0] = xs[:, :F]
        xs2_ref[1] = xs[:, F:]

    return pl.pallas_call(
        body,
        grid=(_GRID,),
        in_specs=[
            pl.BlockSpec((NC, _RB), lambda b: (0, b)),
            pl.BlockSpec((_RB, D_IN), lambda b: (b, 0)),
        ],
        out_specs=[
            pl.BlockSpec((NC, _RB, F), lambda b: (0, b, 0)),
            pl.BlockSpec((_RB, 1), lambda b: (b, 0)),
        ],
        out_shape=[
            jax.ShapeDtypeStruct((NC, NP, F), jnp.float32),
            jax.ShapeDtypeStruct((NP, 1), jnp.float32),
        ],
    )(degp, xpad)


def _tc_mid(agg1, dinv, W1, b1, W2):
    def body(agg_ref, dinv_ref, w1_ref, b1_ref, w2_ref, out_ref):
        di = dinv_ref[...]
        a = jnp.concatenate([agg_ref[0], agg_ref[1]], axis=1) * di
        h = jnp.dot(a, w1_ref[...], preferred_element_type=jnp.float32)
        h = jnp.maximum(h + b1_ref[...][None, :], 0.0)
        m = jnp.dot(h, w2_ref[...], preferred_element_type=jnp.float32) * di
        out_ref[0] = m[:, :F]
        out_ref[1] = m[:, F:]

    return pl.pallas_call(
        body,
        grid=(_GRID,),
        in_specs=[
            pl.BlockSpec((NC, _RB, F), lambda b: (0, b, 0)),
            pl.BlockSpec((_RB, 1), lambda b: (b, 0)),
            pl.BlockSpec((D_IN, HID), lambda b: (0, 0)),
            pl.BlockSpec((HID,), lambda b: (0,)),
            pl.BlockSpec((HID, HID2), lambda b: (0, 0)),
        ],
        out_specs=pl.BlockSpec((NC, _RB, F), lambda b: (0, b, 0)),
        out_shape=jax.ShapeDtypeStruct((NC, NP, F), jnp.float32),
    )(agg1, dinv, W1, b1, W2)


def _tc_final(agg2, dinv, b2, Wc, bc):
    def body(agg_ref, dinv_ref, b2_ref, wc_ref, bc_ref, out_ref):
        di = dinv_ref[...]
        a = jnp.concatenate([agg_ref[0], agg_ref[1]], axis=1) * di
        z = jnp.maximum(a + b2_ref[...][None, :], 0.0)
        logits = jnp.dot(z, wc_ref[...], preferred_element_type=jnp.float32)
        logits = logits + bc_ref[...][None, :]
        m = jnp.max(logits, axis=1, keepdims=True)
        lse = m + jnp.log(jnp.sum(jnp.exp(logits - m), axis=1, keepdims=True))
        out_ref[...] = logits - lse

    return pl.pallas_call(
        body,
        grid=(_GRID,),
        in_specs=[
            pl.BlockSpec((NC, _RB, F), lambda b: (0, b, 0)),
            pl.BlockSpec((_RB, 1), lambda b: (b, 0)),
            pl.BlockSpec((HID2,), lambda b: (0,)),
            pl.BlockSpec((HID2, NCLS), lambda b: (0, 0)),
            pl.BlockSpec((NCLS,), lambda b: (0,)),
        ],
        out_specs=pl.BlockSpec((_RB, NCLS), lambda b: (b, 0)),
        out_shape=jax.ShapeDtypeStruct((NP, NCLS), jnp.float32),
    )(agg2, dinv, b2, Wc, bc)


def kernel(x, edge_index, W1, b1, W2, b2, Wc, bc):
    src = edge_index[0]
    dst = edge_index[1]
    degp = _sc_degree(dst)
    xpad = jnp.pad(x, ((0, NP - N), (0, 0)))
    xs2, dinv = _tc_scale(degp, xpad)
    agg1 = _sc_aggregate(xs2[0], xs2[1], src, dst)
    ms2 = _tc_mid(agg1, dinv, W1, b1, W2)
    agg2 = _sc_aggregate(ms2[0], ms2[1], src, dst)
    out = _tc_final(agg2, dinv, b2, Wc, bc)
    return out[:N]


# speedup vs baseline: 19.5223x; 1.1345x over previous
import functools

import jax
import jax.numpy as jnp
from jax import lax
from jax.experimental import pallas as pl
from jax.experimental.pallas import tpu as pltpu
from jax.experimental.pallas import tpu_sc as plsc

N = 10000
E = 160000
D_IN = 256
HID = 512
HID2 = 256
NCLS = 16

NP = 10240
F = 128
NS = 16
NC = 2
ROWS_PER_TILE = NP // NS
CHUNK = 80
EDGES_PER_TILE = E // NS
DEG_W = NC * NS
DEG_EDGES = E // DEG_W
DEG_CHUNK = 40
DEG_NBUF = 5


def _vsmesh():
    return plsc.VectorSubcoreMesh(core_axis_name="c", subcore_axis_name="s")


def _sc_degree(dst):
    @functools.partial(
        pl.kernel,
        out_type=jax.ShapeDtypeStruct((NC, NP), jnp.float32),
        mesh=_vsmesh(),
        scratch_types=[
            [pltpu.VMEM((DEG_CHUNK,), jnp.int32) for _ in range(DEG_NBUF)],
            pltpu.VMEM((DEG_CHUNK,), jnp.float32),
            pltpu.VMEM((ROWS_PER_TILE,), jnp.float32),
            pltpu.VMEM_SHARED((NP,), jnp.float32),
            [pltpu.SemaphoreType.DMA for _ in range(DEG_NBUF)],
            [pltpu.SemaphoreType.DMA for _ in range(DEG_NBUF)],
        ],
    )
    def k(dst_h, out_h, idxb, ones, zbuf, acc, isem, ssem):
        c = lax.axis_index("c")
        s = lax.axis_index("s")
        wid = s * NC + c

        def fill_z(i, _):
            zbuf[pl.ds(i * 16, 16)] = jnp.zeros((16,), jnp.float32)
            return 0

        lax.fori_loop(0, ROWS_PER_TILE // 16, fill_z, 0)
        one16 = jnp.ones((16,), jnp.float32)
        ones[pl.ds(0, 16)] = one16
        ones[pl.ds(16, 16)] = one16
        ones[pl.ds(24, 16)] = one16

        z0 = s * ROWS_PER_TILE
        pltpu.sync_copy(zbuf, acc.at[pl.ds(z0, ROWS_PER_TILE)])
        plsc.subcore_barrier()

        base = wid * DEG_EDGES
        nchunk = DEG_EDGES // DEG_CHUNK
        ngroup = nchunk // DEG_NBUF

        def load(j, p):
            pltpu.async_copy(dst_h.at[pl.ds(base + j * DEG_CHUNK, DEG_CHUNK)],
                             idxb[p], isem[p])

        def load_wait(p):
            pltpu.make_async_copy(dst_h.at[pl.ds(0, DEG_CHUNK)], idxb[p],
                                  isem[p]).wait()

        def scat_drain(p):
            pltpu.make_async_copy(dst_h.at[pl.ds(0, DEG_CHUNK)], idxb[p],
                                  ssem[p]).wait()

        for p in range(DEG_NBUF):
            load(p, p)

        def group(g, _):
            for p in range(DEG_NBUF):
                load_wait(p)
                pltpu.async_copy(ones, acc.at[idxb[p]], ssem[p], add=True)

            @pl.when(g < ngroup - 1)
            def _():
                for p in range(DEG_NBUF):
                    scat_drain(p)
                    load((g + 1) * DEG_NBUF + p, p)
            return 0

        lax.fori_loop(0, ngroup, group, 0)
        for p in range(DEG_NBUF):
            scat_drain(p)
        plsc.subcore_barrier()
        pltpu.sync_copy(acc.at[pl.ds(z0, ROWS_PER_TILE)],
                        out_h.at[c, pl.ds(z0, ROWS_PER_TILE)])

    return k(dst)


NBUF = 4
NCHUNK = EDGES_PER_TILE // CHUNK
NGROUP = NCHUNK // NBUF
NTAIL = NCHUNK - NGROUP * NBUF


def _sc_aggregate(tlo, thi, src, dst):
    @functools.partial(
        pl.kernel,
        out_type=jax.ShapeDtypeStruct((NC, NP, F), jnp.float32),
        mesh=_vsmesh(),
        scratch_types=[
            [pltpu.VMEM((CHUNK,), jnp.int32) for _ in range(NBUF)],
            [pltpu.VMEM((CHUNK,), jnp.int32) for _ in range(NBUF)],
            [pltpu.VMEM((CHUNK, F), jnp.float32) for _ in range(NBUF)],
            pltpu.VMEM_SHARED((NP, F), jnp.float32),
            [pltpu.SemaphoreType.DMA for _ in range(NBUF)],
            [pltpu.SemaphoreType.DMA for _ in range(NBUF)],
            [pltpu.SemaphoreType.DMA for _ in range(NBUF)],
        ],
    )
    def k(tlo_h, thi_h, src_h, dst_h, out_h, sidx, didx, rows, acc,
          isem, gsem, ssem):
        c = lax.axis_index("c")
        s = lax.axis_index("s")
        r0 = s * ROWS_PER_TILE
        base = s * EDGES_PER_TILE

        def idx_load(j, p):
            pltpu.async_copy(src_h.at[pl.ds(base + j * CHUNK, CHUNK)],
                             sidx[p], isem[p])
            pltpu.async_copy(dst_h.at[pl.ds(base + j * CHUNK, CHUNK)],
                             didx[p], isem[p])

        def idx_wait(p):
            pltpu.make_async_copy(src_h.at[pl.ds(0, CHUNK)], sidx[p],
                                  isem[p]).wait()
            pltpu.make_async_copy(dst_h.at[pl.ds(0, CHUNK)], didx[p],
                                  isem[p]).wait()

        def run(table_h):
            pltpu.sync_copy(table_h.at[pl.ds(r0, ROWS_PER_TILE)],
                            acc.at[pl.ds(r0, ROWS_PER_TILE)])
            plsc.subcore_barrier()
            for p in range(NBUF):
                idx_load(p, p)
            for p in range(NBUF):
                idx_wait(p)
                pltpu.async_copy(table_h.at[sidx[p]], rows[p], gsem[p])

            def scat_drain(p):
                pltpu.make_async_copy(table_h.at[sidx[p]], rows[p],
                                      ssem[p]).wait()

            def group(g, _):
                for p in range(NBUF):
                    pltpu.make_async_copy(table_h.at[sidx[p]], rows[p],
                                          gsem[p]).wait()
                    pltpu.async_copy(rows[p], acc.at[didx[p]], ssem[p],
                                     add=True)

                @pl.when(g < NGROUP - 1)
                def _():
                    for p in range(NBUF):
                        scat_drain(p)
                        idx_load((g + 1) * NBUF + p, p)
                    for p in range(NBUF):
                        idx_wait(p)
                        pltpu.async_copy(table_h.at[sidx[p]], rows[p],
                                         gsem[p])
                return 0

            lax.fori_loop(0, NGROUP, group, 0)
            for p in range(NBUF):
                scat_drain(p)
            for t in range(NTAIL):
                j = NGROUP * NBUF + t
                idx_load(j, 0)
                idx_wait(0)
                pltpu.async_copy(table_h.at[sidx[0]], rows[0], gsem[0]).wait()
                pltpu.sync_copy(rows[0], acc.at[didx[0]], add=True)
            plsc.subcore_barrier()
            pltpu.sync_copy(acc.at[pl.ds(r0, ROWS_PER_TILE)],
                            out_h.at[c, pl.ds(r0, ROWS_PER_TILE)])

        @pl.when(c == 0)
        def _():
            run(tlo_h)

        @pl.when(c == 1)
        def _():
            run(thi_h)

    return k(tlo, thi, src, dst)


_RB = 1024
_GRID = NP // _RB


def _tc_scale(degp, xpad):
    def body(degp_ref, x_ref, xs2_ref, dinv_ref):
        deg = degp_ref[0] + degp_ref[1] + 1.0
        di = lax.rsqrt(deg)
        dinv_ref[...] = di[:, None]
        xs = x_ref[...] * di[:, None]
        xs2_ref[0] = xs[:, :F]
        xs2_ref[1] = xs[:, F:]

    return pl.pallas_call(
        body,
        grid=(_GRID,),
        in_specs=[
            pl.BlockSpec((NC, _RB), lambda b: (0, b)),
            pl.BlockSpec((_RB, D_IN), lambda b: (b, 0)),
        ],
        out_specs=[
            pl.BlockSpec((NC, _RB, F), lambda b: (0, b, 0)),
            pl.BlockSpec((_RB, 1), lambda b: (b, 0)),
        ],
        out_shape=[
            jax.ShapeDtypeStruct((NC, NP, F), jnp.float32),
            jax.ShapeDtypeStruct((NP, 1), jnp.float32),
        ],
    )(degp, xpad)


def _tc_mid(agg1, dinv, W1, b1, W2):
    def body(agg_ref, dinv_ref, w1_ref, b1_ref, w2_ref, out_ref):
        di = dinv_ref[...]
        a = jnp.concatenate([agg_ref[0], agg_ref[1]], axis=1) * di
        h = jnp.dot(a, w1_ref[...], preferred_element_type=jnp.float32)
        h = jnp.maximum(h + b1_ref[...][None, :], 0.0)
        m = jnp.dot(h, w2_ref[...], preferred_element_type=jnp.float32) * di
        out_ref[0] = m[:, :F]
        out_ref[1] = m[:, F:]

    return pl.pallas_call(
        body,
        grid=(_GRID,),
        in_specs=[
            pl.BlockSpec((NC, _RB, F), lambda b: (0, b, 0)),
            pl.BlockSpec((_RB, 1), lambda b: (b, 0)),
            pl.BlockSpec((D_IN, HID), lambda b: (0, 0)),
            pl.BlockSpec((HID,), lambda b: (0,)),
            pl.BlockSpec((HID, HID2), lambda b: (0, 0)),
        ],
        out_specs=pl.BlockSpec((NC, _RB, F), lambda b: (0, b, 0)),
        out_shape=jax.ShapeDtypeStruct((NC, NP, F), jnp.float32),
    )(agg1, dinv, W1, b1, W2)


def _tc_final(agg2, dinv, b2, Wc, bc):
    def body(agg_ref, dinv_ref, b2_ref, wc_ref, bc_ref, out_ref):
        di = dinv_ref[...]
        a = jnp.concatenate([agg_ref[0], agg_ref[1]], axis=1) * di
        z = jnp.maximum(a + b2_ref[...][None, :], 0.0)
        logits = jnp.dot(z, wc_ref[...], preferred_element_type=jnp.float32)
        logits = logits + bc_ref[...][None, :]
        m = jnp.max(logits, axis=1, keepdims=True)
        lse = m + jnp.log(jnp.sum(jnp.exp(logits - m), axis=1, keepdims=True))
        out_ref[...] = logits - lse

    return pl.pallas_call(
        body,
        grid=(_GRID,),
        in_specs=[
            pl.BlockSpec((NC, _RB, F), lambda b: (0, b, 0)),
            pl.BlockSpec((_RB, 1), lambda b: (b, 0)),
            pl.BlockSpec((HID2,), lambda b: (0,)),
            pl.BlockSpec((HID2, NCLS), lambda b: (0, 0)),
            pl.BlockSpec((NCLS,), lambda b: (0,)),
        ],
        out_specs=pl.BlockSpec((_RB, NCLS), lambda b: (b, 0)),
        out_shape=jax.ShapeDtypeStruct((N, NCLS), jnp.float32),
    )(agg2, dinv, b2, Wc, bc)


def kernel(x, edge_index, W1, b1, W2, b2, Wc, bc):
    src = edge_index[0]
    dst = edge_index[1]
    degp = _sc_degree(dst)
    xs2, dinv = _tc_scale(degp, x)
    agg1 = _sc_aggregate(xs2[0], xs2[1], src, dst)
    ms2 = _tc_mid(agg1, dinv, W1, b1, W2)
    agg2 = _sc_aggregate(ms2[0], ms2[1], src, dst)
    return _tc_final(agg2, dinv, b2, Wc, bc)


# speedup vs baseline: 20.0159x; 1.0253x over previous
import functools

import jax
import jax.numpy as jnp
from jax import lax
from jax.experimental import pallas as pl
from jax.experimental.pallas import tpu as pltpu
from jax.experimental.pallas import tpu_sc as plsc

N = 10000
E = 160000
D_IN = 256
HID = 512
HID2 = 256
NCLS = 16

NP = 10240
F = 128
NS = 16
NC = 2
ROWS_PER_TILE = NP // NS
CHUNK = 40
EDGES_PER_TILE = E // NS
DEG_W = NC * NS
DEG_EDGES = E // DEG_W
DEG_CHUNK = 40
DEG_NBUF = 5


def _vsmesh():
    return plsc.VectorSubcoreMesh(core_axis_name="c", subcore_axis_name="s")


def _sc_degree(dst):
    @functools.partial(
        pl.kernel,
        out_type=jax.ShapeDtypeStruct((NC, NP), jnp.float32),
        mesh=_vsmesh(),
        scratch_types=[
            [pltpu.VMEM((DEG_CHUNK,), jnp.int32) for _ in range(DEG_NBUF)],
            pltpu.VMEM((DEG_CHUNK,), jnp.float32),
            pltpu.VMEM((ROWS_PER_TILE,), jnp.float32),
            pltpu.VMEM_SHARED((NP,), jnp.float32),
            [pltpu.SemaphoreType.DMA for _ in range(DEG_NBUF)],
            [pltpu.SemaphoreType.DMA for _ in range(DEG_NBUF)],
        ],
    )
    def k(dst_h, out_h, idxb, ones, zbuf, acc, isem, ssem):
        c = lax.axis_index("c")
        s = lax.axis_index("s")
        wid = s * NC + c

        def fill_z(i, _):
            zbuf[pl.ds(i * 16, 16)] = jnp.zeros((16,), jnp.float32)
            return 0

        lax.fori_loop(0, ROWS_PER_TILE // 16, fill_z, 0)
        one16 = jnp.ones((16,), jnp.float32)
        ones[pl.ds(0, 16)] = one16
        ones[pl.ds(16, 16)] = one16
        ones[pl.ds(24, 16)] = one16

        z0 = s * ROWS_PER_TILE
        pltpu.sync_copy(zbuf, acc.at[pl.ds(z0, ROWS_PER_TILE)])
        plsc.subcore_barrier()

        base = wid * DEG_EDGES
        nchunk = DEG_EDGES // DEG_CHUNK
        ngroup = nchunk // DEG_NBUF

        def load(j, p):
            pltpu.async_copy(dst_h.at[pl.ds(base + j * DEG_CHUNK, DEG_CHUNK)],
                             idxb[p], isem[p])

        def load_wait(p):
            pltpu.make_async_copy(dst_h.at[pl.ds(0, DEG_CHUNK)], idxb[p],
                                  isem[p]).wait()

        def scat_drain(p):
            pltpu.make_async_copy(dst_h.at[pl.ds(0, DEG_CHUNK)], idxb[p],
                                  ssem[p]).wait()

        for p in range(DEG_NBUF):
            load(p, p)

        def group(g, _):
            for p in range(DEG_NBUF):
                load_wait(p)
                pltpu.async_copy(ones, acc.at[idxb[p]], ssem[p], add=True)

            @pl.when(g < ngroup - 1)
            def _():
                for p in range(DEG_NBUF):
                    scat_drain(p)
                    load((g + 1) * DEG_NBUF + p, p)
            return 0

        lax.fori_loop(0, ngroup, group, 0)
        for p in range(DEG_NBUF):
            scat_drain(p)
        plsc.subcore_barrier()
        pltpu.sync_copy(acc.at[pl.ds(z0, ROWS_PER_TILE)],
                        out_h.at[c, pl.ds(z0, ROWS_PER_TILE)])

    return k(dst)


NBUF = 8
NCHUNK = EDGES_PER_TILE // CHUNK
NGROUP = NCHUNK // NBUF
NTAIL = NCHUNK - NGROUP * NBUF


def _sc_aggregate(tlo, thi, src, dst):
    @functools.partial(
        pl.kernel,
        out_type=jax.ShapeDtypeStruct((NC, NP, F), jnp.float32),
        mesh=_vsmesh(),
        scratch_types=[
            [pltpu.VMEM((CHUNK,), jnp.int32) for _ in range(NBUF)],
            [pltpu.VMEM((CHUNK,), jnp.int32) for _ in range(NBUF)],
            [pltpu.VMEM((CHUNK, F), jnp.float32) for _ in range(NBUF)],
            pltpu.VMEM_SHARED((NP, F), jnp.float32),
            [pltpu.SemaphoreType.DMA for _ in range(NBUF)],
            [pltpu.SemaphoreType.DMA for _ in range(NBUF)],
            [pltpu.SemaphoreType.DMA for _ in range(NBUF)],
        ],
    )
    def k(tlo_h, thi_h, src_h, dst_h, out_h, sidx, didx, rows, acc,
          isem, gsem, ssem):
        c = lax.axis_index("c")
        s = lax.axis_index("s")
        r0 = s * ROWS_PER_TILE
        base = s * EDGES_PER_TILE

        def idx_load(j, p):
            pltpu.async_copy(src_h.at[pl.ds(base + j * CHUNK, CHUNK)],
                             sidx[p], isem[p])
            pltpu.async_copy(dst_h.at[pl.ds(base + j * CHUNK, CHUNK)],
                             didx[p], isem[p])

        def idx_wait(p):
            pltpu.make_async_copy(src_h.at[pl.ds(0, CHUNK)], sidx[p],
                                  isem[p]).wait()
            pltpu.make_async_copy(dst_h.at[pl.ds(0, CHUNK)], didx[p],
                                  isem[p]).wait()

        def run(table_h):
            pltpu.sync_copy(table_h.at[pl.ds(r0, ROWS_PER_TILE)],
                            acc.at[pl.ds(r0, ROWS_PER_TILE)])
            plsc.subcore_barrier()
            for p in range(NBUF):
                idx_load(p, p)
            for p in range(NBUF):
                idx_wait(p)
                pltpu.async_copy(table_h.at[sidx[p]], rows[p], gsem[p])

            def scat_drain(p):
                pltpu.make_async_copy(table_h.at[sidx[p]], rows[p],
                                      ssem[p]).wait()

            def group(g, _):
                for p in range(NBUF):
                    pltpu.make_async_copy(table_h.at[sidx[p]], rows[p],
                                          gsem[p]).wait()
                    pltpu.async_copy(rows[p], acc.at[didx[p]], ssem[p],
                                     add=True)

                @pl.when(g < NGROUP - 1)
                def _():
                    for p in range(NBUF):
                        scat_drain(p)
                        idx_load((g + 1) * NBUF + p, p)
                    for p in range(NBUF):
                        idx_wait(p)
                        pltpu.async_copy(table_h.at[sidx[p]], rows[p],
                                         gsem[p])
                return 0

            lax.fori_loop(0, NGROUP, group, 0)
            for p in range(NBUF):
                scat_drain(p)
            for t in range(NTAIL):
                j = NGROUP * NBUF + t
                idx_load(j, 0)
                idx_wait(0)
                pltpu.async_copy(table_h.at[sidx[0]], rows[0], gsem[0]).wait()
                pltpu.sync_copy(rows[0], acc.at[didx[0]], add=True)
            plsc.subcore_barrier()
            pltpu.sync_copy(acc.at[pl.ds(r0, ROWS_PER_TILE)],
                            out_h.at[c, pl.ds(r0, ROWS_PER_TILE)])

        @pl.when(c == 0)
        def _():
            run(tlo_h)

        @pl.when(c == 1)
        def _():
            run(thi_h)

    return k(tlo, thi, src, dst)


_RB = 1024
_GRID = NP // _RB


def _tc_scale(degp, xpad):
    def body(degp_ref, x_ref, xs2_ref, dinv_ref):
        deg = degp_ref[0] + degp_ref[1] + 1.0
        di = lax.rsqrt(deg)
        dinv_ref[...] = di[:, None]
        xs = x_ref[...] * di[:, None]
        xs2_ref[0] = xs[:, :F]
        xs2_ref[1] = xs[:, F:]

    return pl.pallas_call(
        body,
        grid=(_GRID,),
        in_specs=[
            pl.BlockSpec((NC, _RB), lambda b: (0, b)),
            pl.BlockSpec((_RB, D_IN), lambda b: (b, 0)),
        ],
        out_specs=[
            pl.BlockSpec((NC, _RB, F), lambda b: (0, b, 0)),
            pl.BlockSpec((_RB, 1), lambda b: (b, 0)),
        ],
        out_shape=[
            jax.ShapeDtypeStruct((NC, NP, F), jnp.float32),
            jax.ShapeDtypeStruct((NP, 1), jnp.float32),
        ],
    )(degp, xpad)


def _tc_mid(agg1, dinv, W1, b1, W2):
    def body(agg_ref, dinv_ref, w1_ref, b1_ref, w2_ref, out_ref):
        di = dinv_ref[...]
        a = jnp.concatenate([agg_ref[0], agg_ref[1]], axis=1) * di
        h = jnp.dot(a, w1_ref[...], preferred_element_type=jnp.float32)
        h = jnp.maximum(h + b1_ref[...][None, :], 0.0)
        m = jnp.dot(h, w2_ref[...], preferred_element_type=jnp.float32) * di
        out_ref[0] = m[:, :F]
        out_ref[1] = m[:, F:]

    return pl.pallas_call(
        body,
        grid=(_GRID,),
        in_specs=[
            pl.BlockSpec((NC, _RB, F), lambda b: (0, b, 0)),
            pl.BlockSpec((_RB, 1), lambda b: (b, 0)),
            pl.BlockSpec((D_IN, HID), lambda b: (0, 0)),
            pl.BlockSpec((HID,), lambda b: (0,)),
            pl.BlockSpec((HID, HID2), lambda b: (0, 0)),
        ],
        out_specs=pl.BlockSpec((NC, _RB, F), lambda b: (0, b, 0)),
        out_shape=jax.ShapeDtypeStruct((NC, NP, F), jnp.float32),
    )(agg1, dinv, W1, b1, W2)


def _tc_final(agg2, dinv, b2, Wc, bc):
    def body(agg_ref, dinv_ref, b2_ref, wc_ref, bc_ref, out_ref):
        di = dinv_ref[...]
        a = jnp.concatenate([agg_ref[0], agg_ref[1]], axis=1) * di
        z = jnp.maximum(a + b2_ref[...][None, :], 0.0)
        logits = jnp.dot(z, wc_ref[...], preferred_element_type=jnp.float32)
        logits = logits + bc_ref[...][None, :]
        m = jnp.max(logits, axis=1, keepdims=True)
        lse = m + jnp.log(jnp.sum(jnp.exp(logits - m), axis=1, keepdims=True))
        out_ref[...] = logits - lse

    return pl.pallas_call(
        body,
        grid=(_GRID,),
        in_specs=[
            pl.BlockSpec((NC, _RB, F), lambda b: (0, b, 0)),
            pl.BlockSpec((_RB, 1), lambda b: (b, 0)),
            pl.BlockSpec((HID2,), lambda b: (0,)),
            pl.BlockSpec((HID2, NCLS), lambda b: (0, 0)),
            pl.BlockSpec((NCLS,), lambda b: (0,)),
        ],
        out_specs=pl.BlockSpec((_RB, NCLS), lambda b: (b, 0)),
        out_shape=jax.ShapeDtypeStruct((N, NCLS), jnp.float32),
    )(agg2, dinv, b2, Wc, bc)


def kernel(x, edge_index, W1, b1, W2, b2, Wc, bc):
    src = edge_index[0]
    dst = edge_index[1]
    degp = _sc_degree(dst)
    xs2, dinv = _tc_scale(degp, x)
    agg1 = _sc_aggregate(xs2[0], xs2[1], src, dst)
    ms2 = _tc_mid(agg1, dinv, W1, b1, W2)
    agg2 = _sc_aggregate(ms2[0], ms2[1], src, dst)
    return _tc_final(agg2, dinv, b2, Wc, bc)


# speedup vs baseline: 20.8191x; 1.0401x over previous
import functools

import jax
import jax.numpy as jnp
from jax import lax
from jax.experimental import pallas as pl
from jax.experimental.pallas import tpu as pltpu
from jax.experimental.pallas import tpu_sc as plsc

N = 10000
E = 160000
D_IN = 256
HID = 512
HID2 = 256
NCLS = 16

NP = 10240
F = 128
NS = 16
NC = 2
ROWS_PER_TILE = NP // NS
CHUNK = 40
EDGES_PER_TILE = E // NS
DEG_W = NC * NS
DEG_EDGES = E // DEG_W
DEG_CHUNK = 40
DEG_NBUF = 5


def _vsmesh():
    return plsc.VectorSubcoreMesh(core_axis_name="c", subcore_axis_name="s")


def _sc_degree(dst):
    @functools.partial(
        pl.kernel,
        out_type=jax.ShapeDtypeStruct((NC, NP), jnp.float32),
        mesh=_vsmesh(),
        scratch_types=[
            [pltpu.VMEM((DEG_CHUNK,), jnp.int32) for _ in range(DEG_NBUF)],
            pltpu.VMEM((DEG_CHUNK,), jnp.float32),
            pltpu.VMEM((ROWS_PER_TILE,), jnp.float32),
            pltpu.VMEM_SHARED((NP,), jnp.float32),
            [pltpu.SemaphoreType.DMA for _ in range(DEG_NBUF)],
            [pltpu.SemaphoreType.DMA for _ in range(DEG_NBUF)],
        ],
    )
    def k(dst_h, out_h, idxb, ones, zbuf, acc, isem, ssem):
        c = lax.axis_index("c")
        s = lax.axis_index("s")
        wid = s * NC + c

        def fill_z(i, _):
            zbuf[pl.ds(i * 16, 16)] = jnp.zeros((16,), jnp.float32)
            return 0

        lax.fori_loop(0, ROWS_PER_TILE // 16, fill_z, 0)
        one16 = jnp.ones((16,), jnp.float32)
        ones[pl.ds(0, 16)] = one16
        ones[pl.ds(16, 16)] = one16
        ones[pl.ds(24, 16)] = one16

        z0 = s * ROWS_PER_TILE
        pltpu.sync_copy(zbuf, acc.at[pl.ds(z0, ROWS_PER_TILE)])
        plsc.subcore_barrier()

        base = wid * DEG_EDGES
        nchunk = DEG_EDGES // DEG_CHUNK
        ngroup = nchunk // DEG_NBUF

        def load(j, p):
            pltpu.async_copy(dst_h.at[pl.ds(base + j * DEG_CHUNK, DEG_CHUNK)],
                             idxb[p], isem[p])

        def load_wait(p):
            pltpu.make_async_copy(dst_h.at[pl.ds(0, DEG_CHUNK)], idxb[p],
                                  isem[p]).wait()

        def scat_drain(p):
            pltpu.make_async_copy(dst_h.at[pl.ds(0, DEG_CHUNK)], idxb[p],
                                  ssem[p]).wait()

        for p in range(DEG_NBUF):
            load(p, p)

        def group(g, _):
            for p in range(DEG_NBUF):
                load_wait(p)
                pltpu.async_copy(ones, acc.at[idxb[p]], ssem[p], add=True)

            @pl.when(g < ngroup - 1)
            def _():
                for p in range(DEG_NBUF):
                    scat_drain(p)
                    load((g + 1) * DEG_NBUF + p, p)
            return 0

        lax.fori_loop(0, ngroup, group, 0)
        for p in range(DEG_NBUF):
            scat_drain(p)
        plsc.subcore_barrier()
        pltpu.sync_copy(acc.at[pl.ds(z0, ROWS_PER_TILE)],
                        out_h.at[c, pl.ds(z0, ROWS_PER_TILE)])

    return k(dst)


NBUF = 8
NCHUNK = EDGES_PER_TILE // CHUNK
NGROUP = NCHUNK // NBUF
NTAIL = NCHUNK - NGROUP * NBUF


def _sc_aggregate(tlo, thi, src, dst):
    @functools.partial(
        pl.kernel,
        out_type=jax.ShapeDtypeStruct((NC, NP, F), jnp.float32),
        mesh=_vsmesh(),
        scratch_types=[
            [pltpu.VMEM((CHUNK,), jnp.int32) for _ in range(NBUF)],
            [pltpu.VMEM((CHUNK,), jnp.int32) for _ in range(NBUF)],
            [pltpu.VMEM((CHUNK, F), jnp.float32) for _ in range(NBUF)],
            pltpu.VMEM_SHARED((NP, F), jnp.float32),
            [pltpu.SemaphoreType.DMA for _ in range(NBUF)],
            [pltpu.SemaphoreType.DMA for _ in range(NBUF)],
            [pltpu.SemaphoreType.DMA for _ in range(NBUF)],
        ],
    )
    def k(tlo_h, thi_h, src_h, dst_h, out_h, sidx, didx, rows, acc,
          isem, gsem, ssem):
        c = lax.axis_index("c")
        s = lax.axis_index("s")
        r0 = s * ROWS_PER_TILE
        base = s * EDGES_PER_TILE

        def idx_load(j, p):
            pltpu.async_copy(src_h.at[pl.ds(base + j * CHUNK, CHUNK)],
                             sidx[p], isem[p])
            pltpu.async_copy(dst_h.at[pl.ds(base + j * CHUNK, CHUNK)],
                             didx[p], isem[p])

        def idx_wait(p):
            pltpu.make_async_copy(src_h.at[pl.ds(0, CHUNK)], sidx[p],
                                  isem[p]).wait()
            pltpu.make_async_copy(dst_h.at[pl.ds(0, CHUNK)], didx[p],
                                  isem[p]).wait()

        def run(table_h):
            pltpu.sync_copy(table_h.at[pl.ds(r0, ROWS_PER_TILE)],
                            acc.at[pl.ds(r0, ROWS_PER_TILE)])
            plsc.subcore_barrier()
            for p in range(NBUF):
                idx_load(p, p)
            for p in range(NBUF):
                idx_wait(p)
                pltpu.async_copy(table_h.at[sidx[p]], rows[p], gsem[p])

            def scat_drain(p):
                pltpu.make_async_copy(table_h.at[sidx[p]], rows[p],
                                      ssem[p]).wait()

            def group(g, _):
                for p in range(NBUF):
                    pltpu.make_async_copy(table_h.at[sidx[p]], rows[p],
                                          gsem[p]).wait()
                    pltpu.async_copy(rows[p], acc.at[didx[p]], ssem[p],
                                     add=True)

                @pl.when(g < NGROUP - 1)
                def _():
                    for p in range(NBUF):
                        scat_drain(p)
                        idx_load((g + 1) * NBUF + p, p)
                    for p in range(NBUF):
                        idx_wait(p)
                        pltpu.async_copy(table_h.at[sidx[p]], rows[p],
                                         gsem[p])
                return 0

            lax.fori_loop(0, NGROUP, group, 0)
            for p in range(NBUF):
                scat_drain(p)
            for t in range(NTAIL):
                j = NGROUP * NBUF + t
                idx_load(j, 0)
                idx_wait(0)
                pltpu.async_copy(table_h.at[sidx[0]], rows[0], gsem[0]).wait()
                pltpu.sync_copy(rows[0], acc.at[didx[0]], add=True)
            plsc.subcore_barrier()
            pltpu.sync_copy(acc.at[pl.ds(r0, ROWS_PER_TILE)],
                            out_h.at[c, pl.ds(r0, ROWS_PER_TILE)])

        @pl.when(c == 0)
        def _():
            run(tlo_h)

        @pl.when(c == 1)
        def _():
            run(thi_h)

    return k(tlo, thi, src, dst)


_RB = 1024
_GRID = NP // _RB


def _tc_scale(degp, xpad):
    def body(degp_ref, x_ref, lo_ref, hi_ref, dinv_ref):
        deg = degp_ref[0] + degp_ref[1] + 1.0
        di = lax.rsqrt(deg)
        dinv_ref[...] = di[:, None]
        xs = x_ref[...] * di[:, None]
        lo_ref[...] = xs[:, :F]
        hi_ref[...] = xs[:, F:]

    return pl.pallas_call(
        body,
        grid=(_GRID,),
        in_specs=[
            pl.BlockSpec((NC, _RB), lambda b: (0, b)),
            pl.BlockSpec((_RB, D_IN), lambda b: (b, 0)),
        ],
        out_specs=[
            pl.BlockSpec((_RB, F), lambda b: (b, 0)),
            pl.BlockSpec((_RB, F), lambda b: (b, 0)),
            pl.BlockSpec((_RB, 1), lambda b: (b, 0)),
        ],
        out_shape=[
            jax.ShapeDtypeStruct((NP, F), jnp.float32),
            jax.ShapeDtypeStruct((NP, F), jnp.float32),
            jax.ShapeDtypeStruct((NP, 1), jnp.float32),
        ],
    )(degp, xpad)


def _tc_mid(agg1, dinv, W1, b1, W2):
    def body(agg_ref, dinv_ref, w1_ref, b1_ref, w2_ref, lo_ref, hi_ref):
        di = dinv_ref[...]
        a = jnp.concatenate([agg_ref[0], agg_ref[1]], axis=1) * di
        h = jnp.dot(a, w1_ref[...], preferred_element_type=jnp.float32)
        h = jnp.maximum(h + b1_ref[...][None, :], 0.0)
        m = jnp.dot(h, w2_ref[...], preferred_element_type=jnp.float32) * di
        lo_ref[...] = m[:, :F]
        hi_ref[...] = m[:, F:]

    return pl.pallas_call(
        body,
        grid=(_GRID,),
        in_specs=[
            pl.BlockSpec((NC, _RB, F), lambda b: (0, b, 0)),
            pl.BlockSpec((_RB, 1), lambda b: (b, 0)),
            pl.BlockSpec((D_IN, HID), lambda b: (0, 0)),
            pl.BlockSpec((HID,), lambda b: (0,)),
            pl.BlockSpec((HID, HID2), lambda b: (0, 0)),
        ],
        out_specs=[
            pl.BlockSpec((_RB, F), lambda b: (b, 0)),
            pl.BlockSpec((_RB, F), lambda b: (b, 0)),
        ],
        out_shape=[
            jax.ShapeDtypeStruct((NP, F), jnp.float32),
            jax.ShapeDtypeStruct((NP, F), jnp.float32),
        ],
    )(agg1, dinv, W1, b1, W2)


def _tc_final(agg2, dinv, b2, Wc, bc):
    def body(agg_ref, dinv_ref, b2_ref, wc_ref, bc_ref, out_ref):
        di = dinv_ref[...]
        a = jnp.concatenate([agg_ref[0], agg_ref[1]], axis=1) * di
        z = jnp.maximum(a + b2_ref[...][None, :], 0.0)
        logits = jnp.dot(z, wc_ref[...], preferred_element_type=jnp.float32)
        logits = logits + bc_ref[...][None, :]
        m = jnp.max(logits, axis=1, keepdims=True)
        lse = m + jnp.log(jnp.sum(jnp.exp(logits - m), axis=1, keepdims=True))
        out_ref[...] = logits - lse

    return pl.pallas_call(
        body,
        grid=(_GRID,),
        in_specs=[
            pl.BlockSpec((NC, _RB, F), lambda b: (0, b, 0)),
            pl.BlockSpec((_RB, 1), lambda b: (b, 0)),
            pl.BlockSpec((HID2,), lambda b: (0,)),
            pl.BlockSpec((HID2, NCLS), lambda b: (0, 0)),
            pl.BlockSpec((NCLS,), lambda b: (0,)),
        ],
        out_specs=pl.BlockSpec((_RB, NCLS), lambda b: (b, 0)),
        out_shape=jax.ShapeDtypeStruct((N, NCLS), jnp.float32),
    )(agg2, dinv, b2, Wc, bc)


def kernel(x, edge_index, W1, b1, W2, b2, Wc, bc):
    src = edge_index[0]
    dst = edge_index[1]
    degp = _sc_degree(dst)
    xlo, xhi, dinv = _tc_scale(degp, x)
    agg1 = _sc_aggregate(xlo, xhi, src, dst)
    mlo, mhi = _tc_mid(agg1, dinv, W1, b1, W2)
    agg2 = _sc_aggregate(mlo, mhi, src, dst)
    return _tc_final(agg2, dinv, b2, Wc, bc)


# speedup vs baseline: 21.5522x; 1.0352x over previous
import functools

import jax
import jax.numpy as jnp
from jax import lax
from jax.experimental import pallas as pl
from jax.experimental.pallas import tpu as pltpu
from jax.experimental.pallas import tpu_sc as plsc

N = 10000
E = 160000
D_IN = 256
HID = 512
HID2 = 256
NCLS = 16

NP = 10240
F = 128
NS = 16
NC = 2
ROWS_PER_TILE = NP // NS
CHUNK = 40
EDGES_PER_TILE = E // NS
DEG_W = NC * NS
DEG_EDGES = E // DEG_W
DEG_CHUNK = 40
DEG_NBUF = 5


def _vsmesh():
    return plsc.VectorSubcoreMesh(core_axis_name="c", subcore_axis_name="s")


def _sc_degree(dst):
    @functools.partial(
        pl.kernel,
        out_type=jax.ShapeDtypeStruct((NC, NP), jnp.float32),
        mesh=_vsmesh(),
        scratch_types=[
            [pltpu.VMEM((DEG_CHUNK,), jnp.int32) for _ in range(DEG_NBUF)],
            pltpu.VMEM((DEG_CHUNK,), jnp.float32),
            pltpu.VMEM((ROWS_PER_TILE,), jnp.float32),
            pltpu.VMEM_SHARED((NP,), jnp.float32),
            [pltpu.SemaphoreType.DMA for _ in range(DEG_NBUF)],
            [pltpu.SemaphoreType.DMA for _ in range(DEG_NBUF)],
        ],
    )
    def k(dst_h, out_h, idxb, ones, zbuf, acc, isem, ssem):
        c = lax.axis_index("c")
        s = lax.axis_index("s")
        wid = s * NC + c

        def fill_z(i, _):
            zbuf[pl.ds(i * 16, 16)] = jnp.zeros((16,), jnp.float32)
            return 0

        lax.fori_loop(0, ROWS_PER_TILE // 16, fill_z, 0)
        one16 = jnp.ones((16,), jnp.float32)
        ones[pl.ds(0, 16)] = one16
        ones[pl.ds(16, 16)] = one16
        ones[pl.ds(24, 16)] = one16

        z0 = s * ROWS_PER_TILE
        pltpu.sync_copy(zbuf, acc.at[pl.ds(z0, ROWS_PER_TILE)])
        plsc.subcore_barrier()

        base = wid * DEG_EDGES
        nchunk = DEG_EDGES // DEG_CHUNK
        ngroup = nchunk // DEG_NBUF

        def load(j, p):
            pltpu.async_copy(dst_h.at[pl.ds(base + j * DEG_CHUNK, DEG_CHUNK)],
                             idxb[p], isem[p])

        def load_wait(p):
            pltpu.make_async_copy(dst_h.at[pl.ds(0, DEG_CHUNK)], idxb[p],
                                  isem[p]).wait()

        def scat_drain(p):
            pltpu.make_async_copy(dst_h.at[pl.ds(0, DEG_CHUNK)], idxb[p],
                                  ssem[p]).wait()

        for p in range(DEG_NBUF):
            load(p, p)

        def group(g, _):
            for p in range(DEG_NBUF):
                load_wait(p)
                pltpu.async_copy(ones, acc.at[idxb[p]], ssem[p], add=True)

            @pl.when(g < ngroup - 1)
            def _():
                for p in range(DEG_NBUF):
                    scat_drain(p)
                    load((g + 1) * DEG_NBUF + p, p)
            return 0

        lax.fori_loop(0, ngroup, group, 0)
        for p in range(DEG_NBUF):
            scat_drain(p)
        plsc.subcore_barrier()
        pltpu.sync_copy(acc.at[pl.ds(z0, ROWS_PER_TILE)],
                        out_h.at[c, pl.ds(z0, ROWS_PER_TILE)])

    return k(dst)


NBUF = 8
NCHUNK = EDGES_PER_TILE // CHUNK
NGROUP = NCHUNK // NBUF
NTAIL = NCHUNK - NGROUP * NBUF


def _sc_aggregate(tlo, thi, src, dst):
    @functools.partial(
        pl.kernel,
        out_type=jax.ShapeDtypeStruct((NC, NP, F), jnp.float32),
        mesh=_vsmesh(),
        scratch_types=[
            [pltpu.VMEM((CHUNK,), jnp.int32) for _ in range(NBUF)],
            [pltpu.VMEM((CHUNK,), jnp.int32) for _ in range(NBUF)],
            [pltpu.VMEM((CHUNK, F), jnp.float32) for _ in range(NBUF)],
            pltpu.VMEM_SHARED((NP, F), jnp.float32),
            [pltpu.SemaphoreType.DMA for _ in range(NBUF)],
            [pltpu.SemaphoreType.DMA for _ in range(NBUF)],
            [pltpu.SemaphoreType.DMA for _ in range(NBUF)],
            pltpu.SemaphoreType.DMA,
        ],
    )
    def k(tlo_h, thi_h, src_h, dst_h, out_h, sidx, didx, rows, acc,
          isem, gsem, ssem, nsem):
        c = lax.axis_index("c")
        s = lax.axis_index("s")
        r0 = s * ROWS_PER_TILE
        base = s * EDGES_PER_TILE

        def idx_load(j, p):
            pltpu.async_copy(src_h.at[pl.ds(base + j * CHUNK, CHUNK)],
                             sidx[p], isem[p])
            pltpu.async_copy(dst_h.at[pl.ds(base + j * CHUNK, CHUNK)],
                             didx[p], isem[p])

        def idx_wait(p):
            pltpu.make_async_copy(src_h.at[pl.ds(0, CHUNK)], sidx[p],
                                  isem[p]).wait()
            pltpu.make_async_copy(dst_h.at[pl.ds(0, CHUNK)], didx[p],
                                  isem[p]).wait()

        def run(table_h):
            init = pltpu.async_copy(table_h.at[pl.ds(r0, ROWS_PER_TILE)],
                                    acc.at[pl.ds(r0, ROWS_PER_TILE)], nsem)
            for p in range(NBUF):
                idx_load(p, p)
            for p in range(NBUF):
                idx_wait(p)
                pltpu.async_copy(table_h.at[sidx[p]], rows[p], gsem[p])
            init.wait()
            plsc.subcore_barrier()

            def scat_drain(p):
                pltpu.make_async_copy(table_h.at[sidx[p]], rows[p],
                                      ssem[p]).wait()

            def group(g, _):
                for p in range(NBUF):
                    pltpu.make_async_copy(table_h.at[sidx[p]], rows[p],
                                          gsem[p]).wait()
                    pltpu.async_copy(rows[p], acc.at[didx[p]], ssem[p],
                                     add=True)

                @pl.when(g < NGROUP - 1)
                def _():
                    for p in range(NBUF):
                        scat_drain(p)
                        idx_load((g + 1) * NBUF + p, p)
                    for p in range(NBUF):
                        idx_wait(p)
                        pltpu.async_copy(table_h.at[sidx[p]], rows[p],
                                         gsem[p])
                return 0

            lax.fori_loop(0, NGROUP, group, 0)
            for p in range(NBUF):
                scat_drain(p)
            for t in range(NTAIL):
                j = NGROUP * NBUF + t
                idx_load(j, 0)
                idx_wait(0)
                pltpu.async_copy(table_h.at[sidx[0]], rows[0], gsem[0]).wait()
                pltpu.sync_copy(rows[0], acc.at[didx[0]], add=True)
            plsc.subcore_barrier()
            pltpu.sync_copy(acc.at[pl.ds(r0, ROWS_PER_TILE)],
                            out_h.at[c, pl.ds(r0, ROWS_PER_TILE)])

        @pl.when(c == 0)
        def _():
            run(tlo_h)

        @pl.when(c == 1)
        def _():
            run(thi_h)

    return k(tlo, thi, src, dst)


_RB = 2048
_GRID = NP // _RB


def _tc_scale(degp, xpad):
    def body(degp_ref, x_ref, lo_ref, hi_ref, dinv_ref):
        deg = degp_ref[0] + degp_ref[1] + 1.0
        di = lax.rsqrt(deg)
        dinv_ref[...] = di[:, None]
        xs = x_ref[...] * di[:, None]
        lo_ref[...] = xs[:, :F]
        hi_ref[...] = xs[:, F:]

    return pl.pallas_call(
        body,
        grid=(_GRID,),
        in_specs=[
            pl.BlockSpec((NC, _RB), lambda b: (0, b)),
            pl.BlockSpec((_RB, D_IN), lambda b: (b, 0)),
        ],
        out_specs=[
            pl.BlockSpec((_RB, F), lambda b: (b, 0)),
            pl.BlockSpec((_RB, F), lambda b: (b, 0)),
            pl.BlockSpec((_RB, 1), lambda b: (b, 0)),
        ],
        out_shape=[
            jax.ShapeDtypeStruct((NP, F), jnp.float32),
            jax.ShapeDtypeStruct((NP, F), jnp.float32),
            jax.ShapeDtypeStruct((NP, 1), jnp.float32),
        ],
    )(degp, xpad)


def _tc_mid(agg1, dinv, W1, b1, W2):
    def body(agg_ref, dinv_ref, w1_ref, b1_ref, w2_ref, lo_ref, hi_ref):
        di = dinv_ref[...]
        a = jnp.concatenate([agg_ref[0], agg_ref[1]], axis=1) * di
        h = jnp.dot(a, w1_ref[...], preferred_element_type=jnp.float32)
        h = jnp.maximum(h + b1_ref[...][None, :], 0.0)
        m = jnp.dot(h, w2_ref[...], preferred_element_type=jnp.float32) * di
        lo_ref[...] = m[:, :F]
        hi_ref[...] = m[:, F:]

    return pl.pallas_call(
        body,
        grid=(_GRID,),
        in_specs=[
            pl.BlockSpec((NC, _RB, F), lambda b: (0, b, 0)),
            pl.BlockSpec((_RB, 1), lambda b: (b, 0)),
            pl.BlockSpec((D_IN, HID), lambda b: (0, 0)),
            pl.BlockSpec((HID,), lambda b: (0,)),
            pl.BlockSpec((HID, HID2), lambda b: (0, 0)),
        ],
        out_specs=[
            pl.BlockSpec((_RB, F), lambda b: (b, 0)),
            pl.BlockSpec((_RB, F), lambda b: (b, 0)),
        ],
        out_shape=[
            jax.ShapeDtypeStruct((NP, F), jnp.float32),
            jax.ShapeDtypeStruct((NP, F), jnp.float32),
        ],
    )(agg1, dinv, W1, b1, W2)


def _tc_final(agg2, dinv, b2, Wc, bc):
    def body(agg_ref, dinv_ref, b2_ref, wc_ref, bc_ref, out_ref):
        di = dinv_ref[...]
        a = jnp.concatenate([agg_ref[0], agg_ref[1]], axis=1) * di
        z = jnp.maximum(a + b2_ref[...][None, :], 0.0)
        logits = jnp.dot(z, wc_ref[...], preferred_element_type=jnp.float32)
        logits = logits + bc_ref[...][None, :]
        m = jnp.max(logits, axis=1, keepdims=True)
        lse = m + jnp.log(jnp.sum(jnp.exp(logits - m), axis=1, keepdims=True))
        out_ref[...] = logits - lse

    return pl.pallas_call(
        body,
        grid=(_GRID,),
        in_specs=[
            pl.BlockSpec((NC, _RB, F), lambda b: (0, b, 0)),
            pl.BlockSpec((_RB, 1), lambda b: (b, 0)),
            pl.BlockSpec((HID2,), lambda b: (0,)),
            pl.BlockSpec((HID2, NCLS), lambda b: (0, 0)),
            pl.BlockSpec((NCLS,), lambda b: (0,)),
        ],
        out_specs=pl.BlockSpec((_RB, NCLS), lambda b: (b, 0)),
        out_shape=jax.ShapeDtypeStruct((N, NCLS), jnp.float32),
    )(agg2, dinv, b2, Wc, bc)


def kernel(x, edge_index, W1, b1, W2, b2, Wc, bc):
    src = edge_index[0]
    dst = edge_index[1]
    degp = _sc_degree(dst)
    xlo, xhi, dinv = _tc_scale(degp, x)
    agg1 = _sc_aggregate(xlo, xhi, src, dst)
    mlo, mhi = _tc_mid(agg1, dinv, W1, b1, W2)
    agg2 = _sc_aggregate(mlo, mhi, src, dst)
    return _tc_final(agg2, dinv, b2, Wc, bc)
